# baseline (speedup 1.0000x reference)
import functools

import jax
import jax.numpy as jnp
from jax import lax
from jax.experimental import pallas as pl
from jax.experimental.pallas import tpu as pltpu
from jax.experimental.pallas import tpu_sc as plsc

MAXL = 65536
CKEY = 256
CVAL = 256
DELTA = 0.001
KTOP = 50

NC = 2
NS = 16
NW = NC * NS
LANES = 16
RPT = MAXL // NW
CHUNK = 128
NCHUNK = RPT // CHUNK
NVR = RPT // LANES

_f32 = jnp.float32
_i32 = jnp.int32


def _sel(m, x, y):
  return jnp.where(m, x, y)


def _half_clean(aws, ais, bws, bis):
  m = len(aws)
  hw, hi, lw, li = [], [], [], []
  for j in range(m):
    rbw = lax.rev(bws[m - 1 - j], (0,))
    rbi = lax.rev(bis[m - 1 - j], (0,))
    msk = aws[j] >= rbw
    hw.append(_sel(msk, aws[j], rbw))
    hi.append(_sel(msk, ais[j], rbi))
    lw.append(_sel(msk, rbw, aws[j]))
    li.append(_sel(msk, rbi, ais[j]))
  return hw, hi, lw, li


def _bitonic_desc(ws, is_):
  m = len(ws)
  ws = list(ws)
  is_ = list(is_)
  d = m // 2
  while d >= 1:
    for j in range(m):
      if (j % (2 * d)) < d:
        k = j + d
        msk = ws[j] >= ws[k]
        wj = _sel(msk, ws[j], ws[k])
        ij = _sel(msk, is_[j], is_[k])
        wk = _sel(msk, ws[k], ws[j])
        ik = _sel(msk, is_[k], is_[j])
        ws[j], is_[j], ws[k], is_[k] = wj, ij, wk, ik
    d //= 2
  outs = [plsc.sort_key_val(ws[j], is_[j], descending=True) for j in range(m)]
  return [o[0] for o in outs], [o[1] for o in outs]


def _merge_full(aws, ais, bws, bis):
  hw, hi, lw, li = _half_clean(aws, ais, bws, bis)
  hw, hi = _bitonic_desc(hw, hi)
  lw, li = _bitonic_desc(lw, li)
  return hw + lw, hi + li


def _merge_top(aws, ais, bws, bis):
  hw, hi, _, _ = _half_clean(aws, ais, bws, bis)
  return _bitonic_desc(hw, hi)


def _s1_body(key_hbm, keys_hbm, topw_hbm, topi_hbm, part_hbm,
             q_v, buf_v, w_v, aw_v, ai_v, bw_v, bi_v, pw_v):
  cid = lax.axis_index("c")
  sid = lax.axis_index("s")
  wid = sid * NC + cid
  base = wid * RPT

  pltpu.sync_copy(key_hbm, q_v)
  qs = [q_v[pl.ds(LANES * cc, LANES)] for cc in range(CKEY // LANES)]
  iota = lax.iota(_i32, LANES)

  def chunk_body(ci, _):
    pltpu.sync_copy(keys_hbm.at[pl.ds(base + ci * CHUNK, CHUNK), :], buf_v)

    def grp_body(g, _2):
      rowvec = jnp.zeros((LANES,), _f32)
      for k in range(LANES):
        r = g * LANES + k
        accs = [jnp.zeros((LANES,), _f32) for _ in range(4)]
        for cc in range(CKEY // LANES):
          v = buf_v[r, pl.ds(LANES * cc, LANES)]
          d = v - qs[cc]
          accs[cc % 4] = accs[cc % 4] + d * d
        acc = (accs[0] + accs[1]) + (accs[2] + accs[3])
        rowvec = _sel(iota == k, jnp.full((LANES,), jnp.sum(acc)), rowvec)
      w_v[pl.ds(ci * CHUNK + g * LANES, LANES)] = rowvec
      return 0

    lax.fori_loop(0, CHUNK // LANES, grp_body, 0)
    return 0

  lax.fori_loop(0, NCHUNK, chunk_body, 0)

  def wb(j, wacc):
    d = w_v[pl.ds(LANES * j, LANES)]
    w = 1.0 / (d + DELTA)
    w_v[pl.ds(LANES * j, LANES)] = w
    return wacc + w

  wacc = lax.fori_loop(0, NVR, wb, jnp.zeros((LANES,), _f32))
  pw_v[pl.ds(0, LANES)] = wacc

  def r0(j, _):
    w = w_v[pl.ds(LANES * j, LANES)]
    idx = (base + j * LANES) + iota
    sw, si = plsc.sort_key_val(w, idx, descending=True)
    aw_v[pl.ds(LANES * j, LANES)] = sw
    ai_v[pl.ds(LANES * j, LANES)] = si
    return 0

  lax.fori_loop(0, NVR, r0, 0)

  def round_fn(src_w, src_i, dst_w, dst_i, n_pairs, m_in, full):
    def body(p, _):
      off_a = p * (2 * m_in) * LANES
      off_b = off_a + m_in * LANES
      aws = [src_w[pl.ds(off_a + LANES * j, LANES)] for j in range(m_in)]
      ais = [src_i[pl.ds(off_a + LANES * j, LANES)] for j in range(m_in)]
      bws = [src_w[pl.ds(off_b + LANES * j, LANES)] for j in range(m_in)]
      bis = [src_i[pl.ds(off_b + LANES * j, LANES)] for j in range(m_in)]
      if full:
        ow, oi = _merge_full(aws, ais, bws, bis)
        out_off = p * (2 * m_in) * LANES
      else:
        ow, oi = _merge_top(aws, ais, bws, bis)
        out_off = p * m_in * LANES
      for j in range(len(ow)):
        dst_w[pl.ds(out_off + LANES * j, LANES)] = ow[j]
        dst_i[pl.ds(out_off + LANES * j, LANES)] = oi[j]
      return 0

    lax.fori_loop(0, n_pairs, body, 0)

  round_fn(aw_v, ai_v, bw_v, bi_v, 64, 1, True)
  round_fn(bw_v, bi_v, aw_v, ai_v, 32, 2, True)
  round_fn(aw_v, ai_v, bw_v, bi_v, 16, 4, False)
  round_fn(bw_v, bi_v, aw_v, ai_v, 8, 4, False)
  round_fn(aw_v, ai_v, bw_v, bi_v, 4, 4, False)
  round_fn(bw_v, bi_v, aw_v, ai_v, 2, 4, False)
  round_fn(aw_v, ai_v, bw_v, bi_v, 1, 4, False)

  pltpu.sync_copy(bw_v.at[pl.ds(0, 64)], topw_hbm.at[pl.ds(wid * 64, 64)])
  pltpu.sync_copy(bi_v.at[pl.ds(0, 64)], topi_hbm.at[pl.ds(wid * 64, 64)])
  pltpu.sync_copy(pw_v, part_hbm.at[pl.ds(wid * LANES, LANES)])


def _s2_body(topw_hbm, topi_hbm, part_hbm, values_hbm, out_hbm,
             aw_v, ai_v, bw_v, bi_v, part_v, rows_v, idx_v, wsc_v, outv_v, sem):
  cid = lax.axis_index("c")
  sid = lax.axis_index("s")
  wid = sid * NC + cid

  @pl.when(wid == 0)
  def _():
    pltpu.sync_copy(topw_hbm, aw_v)
    pltpu.sync_copy(topi_hbm, ai_v)
    pltpu.sync_copy(part_hbm, part_v)

    def round_fn(src_w, src_i, dst_w, dst_i, n_pairs):
      def body(p, _):
        off_a = 128 * p
        off_b = 128 * p + 64
        aws = [src_w[pl.ds(off_a + LANES * j, LANES)] for j in range(4)]
        ais = [src_i[pl.ds(off_a + LANES * j, LANES)] for j in range(4)]
        bws = [src_w[pl.ds(off_b + LANES * j, LANES)] for j in range(4)]
        bis = [src_i[pl.ds(off_b + LANES * j, LANES)] for j in range(4)]
        ow, oi = _merge_top(aws, ais, bws, bis)
        for j in range(4):
          dst_w[pl.ds(64 * p + LANES * j, LANES)] = ow[j]
          dst_i[pl.ds(64 * p + LANES * j, LANES)] = oi[j]
        return 0

      lax.fori_loop(0, n_pairs, body, 0)

    round_fn(aw_v, ai_v, bw_v, bi_v, 16)
    round_fn(bw_v, bi_v, aw_v, ai_v, 8)
    round_fn(aw_v, ai_v, bw_v, bi_v, 4)
    round_fn(bw_v, bi_v, aw_v, ai_v, 2)
    round_fn(aw_v, ai_v, bw_v, bi_v, 1)

    def ps(j, t):
      return t + part_v[pl.ds(LANES * j, LANES)]

    tvec = lax.fori_loop(0, NW, ps, jnp.zeros((LANES,), _f32))
    inv_total = jnp.ones((LANES,), _f32) / jnp.full((LANES,), jnp.sum(tvec))

    iota = lax.iota(_i32, LANES)
    for j in range(4):
      idx_v[pl.ds(LANES * j, LANES)] = bi_v[pl.ds(LANES * j, LANES)]
      rank = iota + LANES * j
      w = bw_v[pl.ds(LANES * j, LANES)]
      wsc_v[pl.ds(LANES * j, LANES)] = jnp.where(
          rank < KTOP, w * inv_total, jnp.zeros((LANES,), _f32))

    pltpu.async_copy(values_hbm.at[idx_v], rows_v, sem).wait()

    def acc_body(g, accs):
      wgrp = wsc_v[pl.ds(LANES * g, LANES)]
      accs = list(accs)
      for k in range(LANES):
        r = g * LANES + k
        w = wgrp[k]
        for cc in range(CVAL // LANES):
          accs[cc] = accs[cc] + w * rows_v[r, pl.ds(LANES * cc, LANES)]
      return tuple(accs)

    accs = lax.fori_loop(0, 4, acc_body,
                         tuple(jnp.zeros((LANES,), _f32)
                               for _ in range(CVAL // LANES)))
    for cc in range(CVAL // LANES):
      outv_v[pl.ds(LANES * cc, LANES)] = accs[cc]
    pltpu.sync_copy(outv_v, out_hbm)


_mesh = plsc.VectorSubcoreMesh(
    core_axis_name="c", subcore_axis_name="s", num_cores=NC, num_subcores=NS)

_params = pltpu.CompilerParams(needs_layout_passes=False)

_stage1 = functools.partial(
    pl.kernel,
    out_type=[
        jax.ShapeDtypeStruct((NW * 64,), _f32),
        jax.ShapeDtypeStruct((NW * 64,), _i32),
        jax.ShapeDtypeStruct((NW * LANES,), _f32),
    ],
    mesh=_mesh,
    compiler_params=_params,
    scratch_types=[
        pltpu.VMEM((CKEY,), _f32),
        pltpu.VMEM((CHUNK, CKEY), _f32),
        pltpu.VMEM((RPT,), _f32),
        pltpu.VMEM((RPT,), _f32),
        pltpu.VMEM((RPT,), _i32),
        pltpu.VMEM((RPT,), _f32),
        pltpu.VMEM((RPT,), _i32),
        pltpu.VMEM((LANES,), _f32),
    ],
)(_s1_body)

_stage2 = functools.partial(
    pl.kernel,
    out_type=jax.ShapeDtypeStruct((CVAL,), _f32),
    mesh=_mesh,
    compiler_params=_params,
    scratch_types=[
        pltpu.VMEM((NW * 64,), _f32),
        pltpu.VMEM((NW * 64,), _i32),
        pltpu.VMEM((NW * 64,), _f32),
        pltpu.VMEM((NW * 64,), _i32),
        pltpu.VMEM((NW * LANES,), _f32),
        pltpu.VMEM((64, CVAL), _f32),
        pltpu.VMEM((64,), _i32),
        pltpu.VMEM((64,), _f32),
        pltpu.VMEM((CVAL,), _f32),
        pltpu.SemaphoreType.DMA,
    ],
)(_s2_body)


def kernel(key, keys, values):
  topw, topi, part = _stage1(key, keys)
  return _stage2(topw, topi, part, values).reshape(1, CVAL)
